# Initial kernel scaffold; baseline (speedup 1.0000x reference)
import jax
import jax.numpy as jnp
from jax.experimental import pallas as pl


def kernel(x0, x1, x2, gcn_w0, gcn_w1, Wih0, Whh0, bih0, bhh0, Wih1, Whh1, bih1, bhh1, mlp_w1, mlp_b1, mlp_w2, mlp_b2, edge_index0, edge_index1, edge_index2):
    raise NotImplementedError("write your pallas kernel here")


# speedup vs baseline: 1.6706x; 1.6706x over previous
import functools

import jax
import jax.numpy as jnp
from jax import lax
from jax.experimental import pallas as pl
from jax.experimental.pallas import tpu as pltpu
from jax.experimental.pallas import tpu_sc as plsc

N = 10000
NP = 10240
E = 160000
EP = 163840
F = 256
HF = 128
NS = 16
CH = 128
NCHUNK = EP // NS // CH
RPT = NP // NS
BLK = 1024
SLOPE = (1.0 / 8.0 + 1.0 / 3.0) / 2.0

@functools.cache
def _mesh():
    return plsc.VectorSubcoreMesh(
        core_axis_name="c", subcore_axis_name="s", num_cores=2,
        num_subcores=NS)


DW = 16


def _deg_body(src3, dst3, ones_hbm, zeros_hbm, dego_hbm, degi_hbm,
              sidx_v, didx_v, ones_v, acc_sh):
    cid = lax.axis_index("c")
    sid = lax.axis_index("s")
    rows = pl.ds(sid * RPT, RPT)

    pltpu.sync_copy(src3.at[sid], sidx_v)
    pltpu.sync_copy(dst3.at[sid], didx_v)
    pltpu.sync_copy(ones_hbm, ones_v)
    pltpu.sync_copy(zeros_hbm.at[rows], acc_sh.at[rows])
    plsc.subcore_barrier()

    def chunk_s(j, carry):
        pltpu.sync_copy(ones_v, acc_sh.at[sidx_v.at[j]], add=True)
        return carry

    def chunk_d(j, carry):
        pltpu.sync_copy(ones_v, acc_sh.at[didx_v.at[j]], add=True)
        return carry

    @pl.when(cid == 0)
    def _():
        lax.fori_loop(0, NCHUNK, chunk_s, 0)

    @pl.when(cid == 1)
    def _():
        lax.fori_loop(0, NCHUNK, chunk_d, 0)

    plsc.subcore_barrier()

    @pl.when(cid == 0)
    def _():
        pltpu.sync_copy(acc_sh.at[rows], dego_hbm.at[rows])

    @pl.when(cid == 1)
    def _():
        pltpu.sync_copy(acc_sh.at[rows], degi_hbm.at[rows])


@functools.cache
def _deg_call():
    return pl.kernel(
        _deg_body,
        out_type=(jax.ShapeDtypeStruct((NP, DW), jnp.float32),
                  jax.ShapeDtypeStruct((NP, DW), jnp.float32)),
        mesh=_mesh(),
        scratch_types=(
            pltpu.VMEM((NCHUNK, CH), jnp.int32),
            pltpu.VMEM((NCHUNK, CH), jnp.int32),
            pltpu.VMEM((CH, DW), jnp.float32),
            pltpu.VMEM_SHARED((NP, DW), jnp.float32),
        ),
    )


def _agg_body(hl_hbm, hr_hbm, src3, dst3, zeros_hbm, outl_hbm, outr_hbm,
              sidx_v, didx_v, rows_v, acc_sh, sem):
    cid = lax.axis_index("c")
    sid = lax.axis_index("s")
    rows = pl.ds(sid * RPT, RPT)

    pltpu.sync_copy(src3.at[sid], sidx_v)
    pltpu.sync_copy(dst3.at[sid], didx_v)
    pltpu.sync_copy(zeros_hbm.at[rows], acc_sh.at[rows])
    plsc.subcore_barrier()

    def chunk_l(j, carry):
        pltpu.async_copy(hl_hbm.at[sidx_v.at[j]], rows_v, sem).wait()
        pltpu.sync_copy(rows_v, acc_sh.at[didx_v.at[j]], add=True)
        return carry

    def chunk_r(j, carry):
        pltpu.async_copy(hr_hbm.at[sidx_v.at[j]], rows_v, sem).wait()
        pltpu.sync_copy(rows_v, acc_sh.at[didx_v.at[j]], add=True)
        return carry

    @pl.when(cid == 0)
    def _():
        lax.fori_loop(0, NCHUNK, chunk_l, 0)

    @pl.when(cid == 1)
    def _():
        lax.fori_loop(0, NCHUNK, chunk_r, 0)

    plsc.subcore_barrier()

    @pl.when(cid == 0)
    def _():
        pltpu.sync_copy(acc_sh.at[rows], outl_hbm.at[rows])

    @pl.when(cid == 1)
    def _():
        pltpu.sync_copy(acc_sh.at[rows], outr_hbm.at[rows])


@functools.cache
def _agg_call():
    return pl.kernel(
        _agg_body,
        out_type=(jax.ShapeDtypeStruct((NP, HF), jnp.float32),
                  jax.ShapeDtypeStruct((NP, HF), jnp.float32)),
        mesh=_mesh(),
        scratch_types=(
            pltpu.VMEM((NCHUNK, CH), jnp.int32),
            pltpu.VMEM((NCHUNK, CH), jnp.int32),
            pltpu.VMEM((CH, HF), jnp.float32),
            pltpu.VMEM_SHARED((NP, HF), jnp.float32),
            pltpu.SemaphoreType.DMA,
        ),
    )


def _sigmoid(x):
    return 1.0 / (1.0 + jnp.exp(-x))


def _lstm_tc(w0_ref, wt0_ref, bi0_ref, bh0_ref, w1_ref, wt1_ref, bi1_ref,
             bh1_ref, o0_ref, o1_ref):
    for w_ref, wt_ref, bi_ref, bh_ref, o_ref in (
            (w0_ref, wt0_ref, bi0_ref, bh0_ref, o0_ref),
            (w1_ref, wt1_ref, bi1_ref, bh1_ref, o1_ref)):
        gates = (jnp.dot(w_ref[...], wt_ref[...],
                         preferred_element_type=jnp.float32)
                 + bi_ref[...] + bh_ref[...])
        H = F
        gi = _sigmoid(gates[:, 0 * H:1 * H])
        gg = jnp.tanh(gates[:, 2 * H:3 * H])
        go = _sigmoid(gates[:, 3 * H:4 * H])
        o_ref[...] = go * jnp.tanh(gi * gg)


def _scale_tc(x_ref, dego_ref, hl_ref, hr_ref):
    no = lax.rsqrt(jnp.maximum(dego_ref[...], 1.0))
    h = x_ref[...] * no
    hl_ref[...] = h[:, :HF]
    hr_ref[...] = h[:, HF:]


def _mid_tc(al_ref, ar_ref, degi_ref, dego_ref, w_ref, hl_ref, hr_ref):
    ni = lax.rsqrt(jnp.maximum(degi_ref[...], 1.0))
    no = lax.rsqrt(jnp.maximum(dego_ref[...], 1.0))
    agg = jnp.concatenate([al_ref[...], ar_ref[...]], axis=1) * ni
    h = jnp.dot(agg, w_ref[...], preferred_element_type=jnp.float32)
    h = jnp.where(h >= 0, h, h * SLOPE) * no
    hl_ref[...] = h[:, :HF]
    hr_ref[...] = h[:, HF:]


def _post_tc(al_ref, ar_ref, degi_ref, w_ref, mw1_ref, mb1_ref, mw2_ref,
             mb2_ref, out_ref):
    ni = lax.rsqrt(jnp.maximum(degi_ref[...], 1.0))
    agg = jnp.concatenate([al_ref[...], ar_ref[...]], axis=1) * ni
    h = jnp.dot(agg, w_ref[...], preferred_element_type=jnp.float32)
    h = jnp.where(h >= 0, h, h * SLOPE)
    h = jnp.maximum(jnp.dot(h, mw1_ref[...],
                            preferred_element_type=jnp.float32)
                    + mb1_ref[...], 0.0)
    out_ref[...] = jnp.dot(h, mw2_ref[...],
                           preferred_element_type=jnp.float32) + mb2_ref[...]


_row = lambda bs: pl.BlockSpec(bs, lambda i: (i, 0))
_rep = lambda bs: pl.BlockSpec(bs, lambda i: (0, 0))
_rep1 = lambda n: pl.BlockSpec((n,), lambda i: (0,))

_lstm_call = pl.pallas_call(
    _lstm_tc,
    out_shape=(jax.ShapeDtypeStruct((F, F), jnp.float32),
               jax.ShapeDtypeStruct((F, F), jnp.float32)),
)

_scale_call = pl.pallas_call(
    _scale_tc,
    grid=(NP // BLK,),
    in_specs=[_row((BLK, F)), _row((BLK, 1))],
    out_specs=(_row((BLK, HF)), _row((BLK, HF))),
    out_shape=(jax.ShapeDtypeStruct((NP, HF), jnp.float32),
               jax.ShapeDtypeStruct((NP, HF), jnp.float32)),
)

_mid_call = pl.pallas_call(
    _mid_tc,
    grid=(NP // BLK,),
    in_specs=[_row((BLK, HF)), _row((BLK, HF)), _row((BLK, 1)),
              _row((BLK, 1)), _rep((F, F))],
    out_specs=(_row((BLK, HF)), _row((BLK, HF))),
    out_shape=(jax.ShapeDtypeStruct((NP, HF), jnp.float32),
               jax.ShapeDtypeStruct((NP, HF), jnp.float32)),
)

M = 307
C = 2
_post_call = pl.pallas_call(
    _post_tc,
    grid=(NP // BLK,),
    in_specs=[_row((BLK, HF)), _row((BLK, HF)), _row((BLK, 1)), _rep((F, F)),
              _rep((F, M)), _rep1(M), _rep((M, C)), _rep1(C)],
    out_specs=_row((BLK, C)),
    out_shape=jax.ShapeDtypeStruct((NP, C), jnp.float32),
)


def kernel(x0, x1, x2, gcn_w0, gcn_w1, Wih0, Whh0, bih0, bhh0, Wih1, Whh1,
           bih1, bhh1, mlp_w1, mlp_b1, mlp_w2, mlp_b2, edge_index0,
           edge_index1, edge_index2):
    src = edge_index2[0]
    dst = edge_index2[1]
    pad = jnp.full((EP - E,), N, jnp.int32)
    src3 = jnp.concatenate([src.astype(jnp.int32), pad]).reshape(NS, NCHUNK, CH)
    dst3 = jnp.concatenate([dst.astype(jnp.int32), pad]).reshape(NS, NCHUNK, CH)
    xp = jnp.zeros((NP, F), jnp.float32).at[:N].set(x2)
    ones_col = jnp.ones((CH, DW), jnp.float32)
    zeros_col = jnp.zeros((NP, DW), jnp.float32)
    zeros_half = jnp.zeros((NP, HF), jnp.float32)

    ones_half = jnp.ones((NP, HF), jnp.float32)
    degi_full, _ = _agg_call()(ones_half, ones_half, src3, dst3, zeros_half)
    dego_full, _ = _agg_call()(ones_half, ones_half, dst3, src3, zeros_half)
    dego = dego_full[:, :1]
    degi = degi_full[:, :1]
    _ = (ones_col, zeros_col)
    w0, w1 = _lstm_call(gcn_w0, Wih0.T, bih0, bhh0, gcn_w1, Wih1.T, bih1,
                        bhh1)
    hl, hr = _scale_call(xp, dego)
    al, ar = _agg_call()(hl, hr, src3, dst3, zeros_half)
    hl2, hr2 = _mid_call(al, ar, degi, dego, w0)
    al2, ar2 = _agg_call()(hl2, hr2, src3, dst3, zeros_half)
    out = _post_call(al2, ar2, degi, w1, mlp_w1, mlp_b1, mlp_w2, mlp_b2)
    return out[:N]


# speedup vs baseline: 2.3043x; 1.3794x over previous
import functools

import jax
import jax.numpy as jnp
from jax import lax
from jax.experimental import pallas as pl
from jax.experimental.pallas import tpu as pltpu
from jax.experimental.pallas import tpu_sc as plsc

N = 10000
NP = 10240
E = 160000
EP = 163840
F = 256
HF = 128
NS = 16
CH = 128
NCHUNK = EP // NS // CH
RPT = NP // NS
BLK = 1024
SLOPE = (1.0 / 8.0 + 1.0 / 3.0) / 2.0

@functools.cache
def _mesh():
    return plsc.VectorSubcoreMesh(
        core_axis_name="c", subcore_axis_name="s", num_cores=2,
        num_subcores=NS)


NHALF = NCHUNK // 2


def _agg_body(hl_hbm, hr_hbm, idxc_hbm, zeros_hbm, outl_hbm, outr_hbm,
              idx_v, rows0_v, rows1_v, acc_sh, sem0, sem1):
    cid = lax.axis_index("c")
    sid = lax.axis_index("s")
    rows = pl.ds(sid * RPT, RPT)

    pltpu.sync_copy(zeros_hbm.at[rows], acc_sh.at[rows])
    plsc.subcore_barrier()

    def run(h_hbm):
        def do_half(h, carry):
            pltpu.sync_copy(idxc_hbm.at[cid, sid, h], idx_v)
            pltpu.async_copy(h_hbm.at[idx_v.at[0, 0]], rows0_v, sem0)

            def body(i, carry2):
                j0 = 2 * i
                j1 = j0 + 1
                pltpu.make_async_copy(h_hbm.at[idx_v.at[j0, 0]], rows0_v,
                                      sem0).wait()
                pltpu.async_copy(h_hbm.at[idx_v.at[j1, 0]], rows1_v, sem1)
                pltpu.sync_copy(rows0_v, acc_sh.at[idx_v.at[j0, 1]], add=True)
                pltpu.make_async_copy(h_hbm.at[idx_v.at[j1, 0]], rows1_v,
                                      sem1).wait()

                @pl.when(j1 + 1 < NHALF)
                def _():
                    pltpu.async_copy(h_hbm.at[idx_v.at[j1 + 1, 0]], rows0_v,
                                     sem0)

                pltpu.sync_copy(rows1_v, acc_sh.at[idx_v.at[j1, 1]], add=True)
                return carry2

            lax.fori_loop(0, NHALF // 2, body, 0)
            return carry

        lax.fori_loop(0, 2, do_half, 0)

    @pl.when(cid == 0)
    def _():
        run(hl_hbm)

    @pl.when(cid == 1)
    def _():
        run(hr_hbm)

    plsc.subcore_barrier()

    @pl.when(cid == 0)
    def _():
        pltpu.sync_copy(acc_sh.at[rows], outl_hbm.at[rows])

    @pl.when(cid == 1)
    def _():
        pltpu.sync_copy(acc_sh.at[rows], outr_hbm.at[rows])


@functools.cache
def _agg_call():
    return pl.kernel(
        _agg_body,
        out_type=(jax.ShapeDtypeStruct((NP, HF), jnp.float32),
                  jax.ShapeDtypeStruct((NP, HF), jnp.float32)),
        mesh=_mesh(),
        scratch_types=(
            pltpu.VMEM((NHALF, 2, CH), jnp.int32),
            pltpu.VMEM((CH, HF), jnp.float32),
            pltpu.VMEM((CH, HF), jnp.float32),
            pltpu.VMEM_SHARED((NP, HF), jnp.float32),
            pltpu.SemaphoreType.DMA,
            pltpu.SemaphoreType.DMA,
        ),
    )


def _sigmoid(x):
    return 1.0 / (1.0 + jnp.exp(-x))


def _lstm_tc(w0_ref, wt0_ref, bi0_ref, bh0_ref, w1_ref, wt1_ref, bi1_ref,
             bh1_ref, o0_ref, o1_ref):
    for w_ref, wt_ref, bi_ref, bh_ref, o_ref in (
            (w0_ref, wt0_ref, bi0_ref, bh0_ref, o0_ref),
            (w1_ref, wt1_ref, bi1_ref, bh1_ref, o1_ref)):
        gates = (jnp.dot(w_ref[...], wt_ref[...],
                         preferred_element_type=jnp.float32)
                 + bi_ref[...] + bh_ref[...])
        H = F
        gi = _sigmoid(gates[:, 0 * H:1 * H])
        gg = jnp.tanh(gates[:, 2 * H:3 * H])
        go = _sigmoid(gates[:, 3 * H:4 * H])
        o_ref[...] = go * jnp.tanh(gi * gg)


def _scale_tc(x_ref, dego_ref, hl_ref, hr_ref):
    no = lax.rsqrt(jnp.maximum(dego_ref[...], 1.0))
    h = x_ref[...] * no
    hl_ref[...] = h[:, :HF]
    hr_ref[...] = h[:, HF:]


def _mid_tc(al_ref, ar_ref, degi_ref, dego_ref, w_ref, hl_ref, hr_ref):
    ni = lax.rsqrt(jnp.maximum(degi_ref[...], 1.0))
    no = lax.rsqrt(jnp.maximum(dego_ref[...], 1.0))
    agg = jnp.concatenate([al_ref[...], ar_ref[...]], axis=1) * ni
    h = jnp.dot(agg, w_ref[...], preferred_element_type=jnp.float32)
    h = jnp.where(h >= 0, h, h * SLOPE) * no
    hl_ref[...] = h[:, :HF]
    hr_ref[...] = h[:, HF:]


def _post_tc(al_ref, ar_ref, degi_ref, w_ref, mw1_ref, mb1_ref, mw2_ref,
             mb2_ref, out_ref):
    ni = lax.rsqrt(jnp.maximum(degi_ref[...], 1.0))
    agg = jnp.concatenate([al_ref[...], ar_ref[...]], axis=1) * ni
    h = jnp.dot(agg, w_ref[...], preferred_element_type=jnp.float32)
    h = jnp.where(h >= 0, h, h * SLOPE)
    h = jnp.maximum(jnp.dot(h, mw1_ref[...],
                            preferred_element_type=jnp.float32)
                    + mb1_ref[...], 0.0)
    out_ref[...] = jnp.dot(h, mw2_ref[...],
                           preferred_element_type=jnp.float32) + mb2_ref[...]


_row = lambda bs: pl.BlockSpec(bs, lambda i: (i, 0))
_rep = lambda bs: pl.BlockSpec(bs, lambda i: (0, 0))
_rep1 = lambda n: pl.BlockSpec((n,), lambda i: (0,))

_lstm_call = pl.pallas_call(
    _lstm_tc,
    out_shape=(jax.ShapeDtypeStruct((F, F), jnp.float32),
               jax.ShapeDtypeStruct((F, F), jnp.float32)),
)

_scale_call = pl.pallas_call(
    _scale_tc,
    grid=(NP // BLK,),
    in_specs=[_row((BLK, F)), _row((BLK, 1))],
    out_specs=(_row((BLK, HF)), _row((BLK, HF))),
    out_shape=(jax.ShapeDtypeStruct((NP, HF), jnp.float32),
               jax.ShapeDtypeStruct((NP, HF), jnp.float32)),
)

_mid_call = pl.pallas_call(
    _mid_tc,
    grid=(NP // BLK,),
    in_specs=[_row((BLK, HF)), _row((BLK, HF)), _row((BLK, 1)),
              _row((BLK, 1)), _rep((F, F))],
    out_specs=(_row((BLK, HF)), _row((BLK, HF))),
    out_shape=(jax.ShapeDtypeStruct((NP, HF), jnp.float32),
               jax.ShapeDtypeStruct((NP, HF), jnp.float32)),
)

M = 307
C = 2
_post_call = pl.pallas_call(
    _post_tc,
    grid=(NP // BLK,),
    in_specs=[_row((BLK, HF)), _row((BLK, HF)), _row((BLK, 1)), _rep((F, F)),
              _rep((F, M)), _rep1(M), _rep((M, C)), _rep1(C)],
    out_specs=_row((BLK, C)),
    out_shape=jax.ShapeDtypeStruct((NP, C), jnp.float32),
)


def kernel(x0, x1, x2, gcn_w0, gcn_w1, Wih0, Whh0, bih0, bhh0, Wih1, Whh1,
           bih1, bhh1, mlp_w1, mlp_b1, mlp_w2, mlp_b2, edge_index0,
           edge_index1, edge_index2):
    src = edge_index2[0]
    dst = edge_index2[1]
    pad = jnp.full((EP - E,), N, jnp.int32)
    src4 = jnp.concatenate([src.astype(jnp.int32), pad]).reshape(
        NS, 2, NHALF, CH)
    dst4 = jnp.concatenate([dst.astype(jnp.int32), pad]).reshape(
        NS, 2, NHALF, CH)
    conv_pair = jnp.stack([src4, dst4], axis=-2)
    idx_conv = jnp.stack([conv_pair, conv_pair], axis=0)
    idx_deg = jnp.stack([jnp.stack([src4, src4], axis=-2),
                         jnp.stack([dst4, dst4], axis=-2)], axis=0)
    xp = jnp.zeros((NP, F), jnp.float32).at[:N].set(x2)
    zeros_half = jnp.zeros((NP, HF), jnp.float32)
    ones_tab = jnp.ones((NP, HF), jnp.float32)

    dego_full, degi_full = _agg_call()(ones_tab, ones_tab, idx_deg,
                                       zeros_half)
    dego = dego_full[:, :1]
    degi = degi_full[:, :1]
    w0, w1 = _lstm_call(gcn_w0, Wih0.T, bih0, bhh0, gcn_w1, Wih1.T, bih1,
                        bhh1)
    hl, hr = _scale_call(xp, dego)
    al, ar = _agg_call()(hl, hr, idx_conv, zeros_half)
    hl2, hr2 = _mid_call(al, ar, degi, dego, w0)
    al2, ar2 = _agg_call()(hl2, hr2, idx_conv, zeros_half)
    out = _post_call(al2, ar2, degi, w1, mlp_w1, mlp_b1, mlp_w2, mlp_b2)
    return out[:N]


# speedup vs baseline: 3.2393x; 1.4057x over previous
import functools

import jax
import jax.numpy as jnp
from jax import lax
from jax.experimental import pallas as pl
from jax.experimental.pallas import tpu as pltpu
from jax.experimental.pallas import tpu_sc as plsc

N = 10000
NP = 10240
E = 160000
EP = 163840
F = 256
HF = 128
NS = 16
CH = 128
NCHUNK = EP // NS // CH
RPT = NP // NS
BLK = 1024
SLOPE = (1.0 / 8.0 + 1.0 / 3.0) / 2.0

@functools.cache
def _mesh():
    return plsc.VectorSubcoreMesh(
        core_axis_name="c", subcore_axis_name="s", num_cores=2,
        num_subcores=NS)


NHALF = NCHUNK // 2


def _agg_body(hl_hbm, hr_hbm, idxc_hbm, zeros_hbm, outl_hbm, outr_hbm,
              idx_v, rows0_v, rows1_v, acc_sh, sem0, sem1):
    cid = lax.axis_index("c")
    sid = lax.axis_index("s")
    rows = pl.ds(sid * RPT, RPT)

    pltpu.sync_copy(zeros_hbm.at[rows], acc_sh.at[rows])
    plsc.subcore_barrier()

    def run(h_hbm):
        def do_half(h, carry):
            pltpu.sync_copy(idxc_hbm.at[cid, sid, h], idx_v)
            pltpu.async_copy(h_hbm.at[idx_v.at[0, 0]], rows0_v, sem0)

            def body(i, carry2):
                j0 = 2 * i
                j1 = j0 + 1
                pltpu.make_async_copy(h_hbm.at[idx_v.at[j0, 0]], rows0_v,
                                      sem0).wait()
                pltpu.async_copy(h_hbm.at[idx_v.at[j1, 0]], rows1_v, sem1)
                pltpu.sync_copy(rows0_v, acc_sh.at[idx_v.at[j0, 1]], add=True)
                pltpu.make_async_copy(h_hbm.at[idx_v.at[j1, 0]], rows1_v,
                                      sem1).wait()

                @pl.when(j1 + 1 < NHALF)
                def _():
                    pltpu.async_copy(h_hbm.at[idx_v.at[j1 + 1, 0]], rows0_v,
                                     sem0)

                pltpu.sync_copy(rows1_v, acc_sh.at[idx_v.at[j1, 1]], add=True)
                return carry2

            lax.fori_loop(0, NHALF // 2, body, 0)
            return carry

        lax.fori_loop(0, 2, do_half, 0)

    @pl.when(cid == 0)
    def _():
        run(hl_hbm)

    @pl.when(cid == 1)
    def _():
        run(hr_hbm)

    plsc.subcore_barrier()

    @pl.when(cid == 0)
    def _():
        pltpu.sync_copy(acc_sh.at[rows], outl_hbm.at[rows])

    @pl.when(cid == 1)
    def _():
        pltpu.sync_copy(acc_sh.at[rows], outr_hbm.at[rows])


@functools.cache
def _agg_call(dtype=jnp.float32):
    return pl.kernel(
        _agg_body,
        out_type=(jax.ShapeDtypeStruct((NP, HF), dtype),
                  jax.ShapeDtypeStruct((NP, HF), dtype)),
        mesh=_mesh(),
        scratch_types=(
            pltpu.VMEM((NHALF, 2, CH), jnp.int32),
            pltpu.VMEM((CH, HF), dtype),
            pltpu.VMEM((CH, HF), dtype),
            pltpu.VMEM_SHARED((NP, HF), dtype),
            pltpu.SemaphoreType.DMA,
            pltpu.SemaphoreType.DMA,
        ),
    )


def _degc_body(ones_hbm, zeros_hbm, idx2_hbm, dego_hbm, degi_hbm,
               idx_v, ones_v, acc_sh):
    cid = lax.axis_index("c")
    sid = lax.axis_index("s")
    rows = pl.ds(sid * RPT, RPT)

    pltpu.sync_copy(idx2_hbm.at[cid, sid], idx_v)
    pltpu.sync_copy(ones_hbm, ones_v)
    pltpu.sync_copy(zeros_hbm.at[rows], acc_sh.at[rows])
    plsc.subcore_barrier()

    def chunk(j, carry):
        pltpu.sync_copy(ones_v, acc_sh.at[idx_v.at[j]], add=True)
        return carry

    lax.fori_loop(0, NCHUNK, chunk, 0)
    plsc.subcore_barrier()

    @pl.when(cid == 0)
    def _():
        pltpu.sync_copy(acc_sh.at[rows], dego_hbm.at[rows])

    @pl.when(cid == 1)
    def _():
        pltpu.sync_copy(acc_sh.at[rows], degi_hbm.at[rows])


@functools.cache
def _degc_call():
    return pl.kernel(
        _degc_body,
        out_type=(jax.ShapeDtypeStruct((NP, HF), jnp.float32),
                  jax.ShapeDtypeStruct((NP, HF), jnp.float32)),
        mesh=_mesh(),
        scratch_types=(
            pltpu.VMEM((NCHUNK, CH), jnp.int32),
            pltpu.VMEM((CH, HF), jnp.float32),
            pltpu.VMEM_SHARED((NP, HF), jnp.float32),
        ),
    )


def _sigmoid(x):
    return 1.0 / (1.0 + jnp.exp(-x))


def _lstm_tc(w0_ref, wt0_ref, bi0_ref, bh0_ref, w1_ref, wt1_ref, bi1_ref,
             bh1_ref, o0_ref, o1_ref):
    for w_ref, wt_ref, bi_ref, bh_ref, o_ref in (
            (w0_ref, wt0_ref, bi0_ref, bh0_ref, o0_ref),
            (w1_ref, wt1_ref, bi1_ref, bh1_ref, o1_ref)):
        gates = (jnp.dot(w_ref[...], wt_ref[...],
                         preferred_element_type=jnp.float32)
                 + bi_ref[...] + bh_ref[...])
        H = F
        gi = _sigmoid(gates[:, 0 * H:1 * H])
        gg = jnp.tanh(gates[:, 2 * H:3 * H])
        go = _sigmoid(gates[:, 3 * H:4 * H])
        o_ref[...] = go * jnp.tanh(gi * gg)


def _scale_tc(x_ref, dego_ref, hl_ref, hr_ref):
    no = lax.rsqrt(jnp.maximum(dego_ref[...], 1.0))
    h = x_ref[...] * no
    hl_ref[...] = h[:, :HF]
    hr_ref[...] = h[:, HF:]


def _mid_tc(al_ref, ar_ref, degi_ref, dego_ref, w_ref, hl_ref, hr_ref):
    ni = lax.rsqrt(jnp.maximum(degi_ref[...], 1.0))
    no = lax.rsqrt(jnp.maximum(dego_ref[...], 1.0))
    agg = jnp.concatenate([al_ref[...], ar_ref[...]], axis=1) * ni
    h = jnp.dot(agg, w_ref[...], preferred_element_type=jnp.float32)
    h = jnp.where(h >= 0, h, h * SLOPE) * no
    hl_ref[...] = h[:, :HF]
    hr_ref[...] = h[:, HF:]


def _post_tc(al_ref, ar_ref, degi_ref, w_ref, mw1_ref, mb1_ref, mw2_ref,
             mb2_ref, out_ref):
    ni = lax.rsqrt(jnp.maximum(degi_ref[...], 1.0))
    agg = jnp.concatenate([al_ref[...], ar_ref[...]], axis=1) * ni
    h = jnp.dot(agg, w_ref[...], preferred_element_type=jnp.float32)
    h = jnp.where(h >= 0, h, h * SLOPE)
    h = jnp.maximum(jnp.dot(h, mw1_ref[...],
                            preferred_element_type=jnp.float32)
                    + mb1_ref[...], 0.0)
    out_ref[...] = jnp.dot(h, mw2_ref[...],
                           preferred_element_type=jnp.float32) + mb2_ref[...]


_row = lambda bs: pl.BlockSpec(bs, lambda i: (i, 0))
_rep = lambda bs: pl.BlockSpec(bs, lambda i: (0, 0))
_rep1 = lambda n: pl.BlockSpec((n,), lambda i: (0,))

_lstm_call = pl.pallas_call(
    _lstm_tc,
    out_shape=(jax.ShapeDtypeStruct((F, F), jnp.float32),
               jax.ShapeDtypeStruct((F, F), jnp.float32)),
)

_scale_call = pl.pallas_call(
    _scale_tc,
    grid=(NP // BLK,),
    in_specs=[_row((BLK, F)), _row((BLK, 1))],
    out_specs=(_row((BLK, HF)), _row((BLK, HF))),
    out_shape=(jax.ShapeDtypeStruct((NP, HF), jnp.float32),
               jax.ShapeDtypeStruct((NP, HF), jnp.float32)),
)

_mid_call = pl.pallas_call(
    _mid_tc,
    grid=(NP // BLK,),
    in_specs=[_row((BLK, HF)), _row((BLK, HF)), _row((BLK, 1)),
              _row((BLK, 1)), _rep((F, F))],
    out_specs=(_row((BLK, HF)), _row((BLK, HF))),
    out_shape=(jax.ShapeDtypeStruct((NP, HF), jnp.float32),
               jax.ShapeDtypeStruct((NP, HF), jnp.float32)),
)

M = 307
C = 2
_post_call = pl.pallas_call(
    _post_tc,
    grid=(NP // BLK,),
    in_specs=[_row((BLK, HF)), _row((BLK, HF)), _row((BLK, 1)), _rep((F, F)),
              _rep((F, M)), _rep1(M), _rep((M, C)), _rep1(C)],
    out_specs=_row((BLK, C)),
    out_shape=jax.ShapeDtypeStruct((NP, C), jnp.float32),
)


def kernel(x0, x1, x2, gcn_w0, gcn_w1, Wih0, Whh0, bih0, bhh0, Wih1, Whh1,
           bih1, bhh1, mlp_w1, mlp_b1, mlp_w2, mlp_b2, edge_index0,
           edge_index1, edge_index2):
    src = edge_index2[0]
    dst = edge_index2[1]
    pad = jnp.full((EP - E,), N, jnp.int32)
    src4 = jnp.concatenate([src.astype(jnp.int32), pad]).reshape(
        NS, 2, NHALF, CH)
    dst4 = jnp.concatenate([dst.astype(jnp.int32), pad]).reshape(
        NS, 2, NHALF, CH)
    conv_pair = jnp.stack([src4, dst4], axis=-2)
    idx_conv = jnp.stack([conv_pair, conv_pair], axis=0)
    idx_deg = jnp.stack([src4, dst4]).reshape(2, NS, NCHUNK, CH)
    xp = jnp.zeros((NP, F), jnp.float32).at[:N].set(x2)
    zeros_half = jnp.zeros((NP, HF), jnp.float32)
    ones_row = jnp.ones((CH, HF), jnp.float32)

    dego_full, degi_full = _degc_call()(ones_row, zeros_half, idx_deg)
    dego = dego_full[:, :1]
    degi = degi_full[:, :1]
    w0, w1 = _lstm_call(gcn_w0, Wih0.T, bih0, bhh0, gcn_w1, Wih1.T, bih1,
                        bhh1)
    hl, hr = _scale_call(xp, dego)
    al, ar = _agg_call()(hl, hr, idx_conv, zeros_half)
    hl2, hr2 = _mid_call(al, ar, degi, dego, w0)
    al2, ar2 = _agg_call()(hl2, hr2, idx_conv, zeros_half)
    out = _post_call(al2, ar2, degi, w1, mlp_w1, mlp_b1, mlp_w2, mlp_b2)
    return out[:N]
